# baseline (speedup 1.0000x reference)
import functools

import jax
import jax.numpy as jnp
from jax import lax
from jax.experimental import pallas as pl
from jax.experimental.pallas import tpu as pltpu
from jax.experimental.pallas import tpu_sc as plsc

N_NODES = 10000
N_PAD = 10240
N_EDGES = 320000
HIDDEN = 128
EDGE_IN = 16

NODE_BLK = 1280
EDGE_BLK = 3200

SC_CHUNK = 3200
SC_WINDOWS = SC_CHUNK // 16
SC_NCHUNKS = N_EDGES // SC_CHUNK
FSLICE = 4


def _encode_body(x_ref, we_ref, be_ref, wm1_ref, h_ref, gt_ref):
    h = jnp.maximum(x_ref[...] @ we_ref[...].T + be_ref[...], 0.0)
    h_ref[...] = h
    gt_ref[...] = wm1_ref[...] @ h.T


def _encode(x, We, be, Wm1):
    grid = (N_PAD // NODE_BLK,)
    return pl.pallas_call(
        _encode_body,
        grid=grid,
        in_specs=[
            pl.BlockSpec((NODE_BLK, HIDDEN), lambda i: (i, 0)),
            pl.BlockSpec((HIDDEN, HIDDEN), lambda i: (0, 0)),
            pl.BlockSpec((1, HIDDEN), lambda i: (0, 0)),
            pl.BlockSpec((HIDDEN, HIDDEN), lambda i: (0, 0)),
        ],
        out_specs=[
            pl.BlockSpec((NODE_BLK, HIDDEN), lambda i: (i, 0)),
            pl.BlockSpec((HIDDEN, NODE_BLK), lambda i: (0, i)),
        ],
        out_shape=[
            jax.ShapeDtypeStruct((N_PAD, HIDDEN), jnp.float32),
            jax.ShapeDtypeStruct((HIDDEN, N_PAD), jnp.float32),
        ],
    )(x, We, be.reshape(1, HIDDEN), Wm1)


def _edge_linear_body(ea_ref, wm2_ref, et_ref):
    et_ref[...] = wm2_ref[...] @ ea_ref[...].T


def _edge_linear(edge_attr, Wm2):
    grid = (N_EDGES // EDGE_BLK,)
    return pl.pallas_call(
        _edge_linear_body,
        grid=grid,
        in_specs=[
            pl.BlockSpec((EDGE_BLK, EDGE_IN), lambda i: (i, 0)),
            pl.BlockSpec((HIDDEN, EDGE_IN), lambda i: (0, 0)),
        ],
        out_specs=pl.BlockSpec((HIDDEN, EDGE_BLK), lambda i: (0, i)),
        out_shape=jax.ShapeDtypeStruct((HIDDEN, N_EDGES), jnp.float32),
    )(edge_attr, Wm2)


def _update_body(h_ref, at_ref, wu1_ref, wu2_ref, bm_ref, bu_ref, wd_ref,
                 bd_ref, o_ref):
    a = at_ref[...].T
    a = jnp.where(a == jnp.inf, 0.0, a + bm_ref[...])
    u = h_ref[...] @ wu1_ref[...].T + a @ wu2_ref[...].T + bu_ref[...]
    d = jnp.sum(u * wd_ref[...], axis=1, keepdims=True) + bd_ref[...]
    o_ref[...] = jax.nn.sigmoid(d)


def _update_decode(h, aggT, Wu1, Wu2, bm, bu, Wd, bd):
    grid = (N_PAD // NODE_BLK,)
    return pl.pallas_call(
        _update_body,
        grid=grid,
        in_specs=[
            pl.BlockSpec((NODE_BLK, HIDDEN), lambda i: (i, 0)),
            pl.BlockSpec((HIDDEN, NODE_BLK), lambda i: (0, i)),
            pl.BlockSpec((HIDDEN, HIDDEN), lambda i: (0, 0)),
            pl.BlockSpec((HIDDEN, HIDDEN), lambda i: (0, 0)),
            pl.BlockSpec((1, HIDDEN), lambda i: (0, 0)),
            pl.BlockSpec((1, HIDDEN), lambda i: (0, 0)),
            pl.BlockSpec((1, HIDDEN), lambda i: (0, 0)),
            pl.BlockSpec((1, 1), lambda i: (0, 0)),
        ],
        out_specs=pl.BlockSpec((NODE_BLK, 1), lambda i: (i, 0)),
        out_shape=jax.ShapeDtypeStruct((N_PAD, 1), jnp.float32),
    )(h, aggT, Wu1, Wu2, bm.reshape(1, HIDDEN), bu.reshape(1, HIDDEN),
      Wd.reshape(1, HIDDEN), bd.reshape(1, 1))


def _sc_aggregate(src, dst, gT, eT):
    mesh = plsc.VectorSubcoreMesh(core_axis_name="c", subcore_axis_name="s")

    @functools.partial(
        pl.kernel,
        mesh=mesh,
        compiler_params=pltpu.CompilerParams(needs_layout_passes=False),
        out_type=jax.ShapeDtypeStruct((HIDDEN, N_PAD), jnp.float32),
        scratch_types=[
            pltpu.VMEM((FSLICE * N_PAD,), jnp.float32),
            pltpu.VMEM((FSLICE * N_PAD,), jnp.float32),
            pltpu.VMEM((SC_CHUNK,), jnp.int32),
            pltpu.VMEM((SC_CHUNK,), jnp.int32),
            pltpu.VMEM((FSLICE * SC_CHUNK,), jnp.float32),
        ],
    )
    def agg(src_hbm, dst_hbm, gt_hbm, et_hbm, out_hbm, g_v, acc_v, s_v, d_v,
            e_v):
        w = lax.axis_index("s") * 2 + lax.axis_index("c")
        frow = w * FSLICE

        iota = lax.iota(jnp.int32, 16)
        pat_edge = iota // 4
        pat_feat = iota - pat_edge * 4
        zero16 = jnp.zeros((16,), jnp.int32)
        inf16 = jnp.full((16,), jnp.inf, jnp.float32)
        c_foff = pat_feat * N_PAD
        c_sd = [4 * j + pat_edge for j in range(4)]
        c_e = [pat_feat * SC_CHUNK + 4 * j + pat_edge for j in range(4)]
        edge_sel = [pat_edge == r for r in range(4)]
        def pair_pat(part):
            pnum = iota + 16 * part
            g = pnum // 6
            q = pnum - g * 6
            a = jnp.where(q < 3, 0, jnp.where(q < 5, 1, 2))
            b = jnp.where(q < 3, q + 1, jnp.where(q < 5, q - 1, 3))
            return jnp.minimum(4 * g + a, 15), jnp.minimum(4 * g + b, 15)
        c_pa1, c_pb1 = pair_pat(0)
        c_pa2, c_pb2 = pair_pat(1)
        m_valid2 = iota < 8

        for r in range(FSLICE):
            pltpu.sync_copy(gt_hbm.at[frow + r, :],
                            g_v.at[pl.ds(r * N_PAD, N_PAD)])

        def init_body(i, _):
            acc_v[pl.ds(i * 16, 16)] = inf16
            return 0

        lax.fori_loop(0, FSLICE * N_PAD // 16, init_body, 0)

        def chunk_body(k, _):
            base = k * SC_CHUNK
            pltpu.sync_copy(src_hbm.at[pl.ds(base, SC_CHUNK)], s_v)
            pltpu.sync_copy(dst_hbm.at[pl.ds(base, SC_CHUNK)], d_v)
            for r in range(FSLICE):
                pltpu.sync_copy(et_hbm.at[frow + r, pl.ds(base, SC_CHUNK)],
                                e_v.at[pl.ds(r * SC_CHUNK, SC_CHUNK)])

            def window_body(t, _):
                tbs = zero16 + t * 16

                a1 = plsc.load_gather(d_v, [tbs + c_pa1])
                b1 = plsc.load_gather(d_v, [tbs + c_pb1])
                a2 = plsc.load_gather(d_v, [tbs + c_pa2])
                b2 = plsc.load_gather(d_v, [tbs + c_pb2])
                ndup = (plsc.all_reduce_population_count(a1 == b1) +
                        plsc.all_reduce_population_count(
                            jnp.logical_and(a2 == b2, m_valid2)))

                def do_groups(serialize):
                    for j in range(4):
                        isd = tbs + c_sd[j]
                        isrc = plsc.load_gather(s_v, [isd])
                        idst = plsc.load_gather(d_v, [isd])
                        ev = plsc.load_gather(e_v, [tbs + c_e[j]])
                        gv = plsc.load_gather(g_v, [c_foff + isrc])
                        m = gv + ev
                        ia = c_foff + idst
                        if serialize:
                            for r in range(4):
                                av = plsc.load_gather(acc_v, [ia])
                                plsc.store_scatter(acc_v, [ia],
                                                   jnp.minimum(av, m),
                                                   mask=edge_sel[r])
                        else:
                            av = plsc.load_gather(acc_v, [ia])
                            plsc.store_scatter(acc_v, [ia], jnp.minimum(av, m))

                lax.cond(jnp.max(ndup) > 0,
                         lambda: do_groups(True),
                         lambda: do_groups(False))
                return 0

            lax.fori_loop(0, SC_WINDOWS, window_body, 0)
            return 0

        lax.fori_loop(0, SC_NCHUNKS, chunk_body, 0)

        for r in range(FSLICE):
            pltpu.sync_copy(acc_v.at[pl.ds(r * N_PAD, N_PAD)],
                            out_hbm.at[frow + r, :])

    return agg(src, dst, gT, eT)


def kernel(x, edge_index, edge_attr, We, be, Wm, bm, Wu, bu, Wd, bd):
    src = edge_index[0].astype(jnp.int32)
    dst = edge_index[1].astype(jnp.int32)
    Wm1 = Wm[:, :HIDDEN]
    Wm2 = Wm[:, HIDDEN:]
    Wu1 = Wu[:, :HIDDEN]
    Wu2 = Wu[:, HIDDEN:]

    x_pad = jnp.pad(x, ((0, N_PAD - N_NODES), (0, 0)))
    h, gT = _encode(x_pad, We, be, Wm1)
    eT = _edge_linear(edge_attr, Wm2)
    aggT = _sc_aggregate(src, dst, gT, eT)
    out = _update_decode(h, aggT, Wu1, Wu2, bm, bu, Wd, bd)
    return out[:N_NODES]


# speedup vs baseline: 1.5834x; 1.5834x over previous
import functools

import jax
import jax.numpy as jnp
from jax import lax
from jax.experimental import pallas as pl
from jax.experimental.pallas import tpu as pltpu
from jax.experimental.pallas import tpu_sc as plsc

N_NODES = 10000
N_PAD = 10240
N_EDGES = 320000
HIDDEN = 128
EDGE_IN = 16

NODE_BLK = 1280
EDGE_BLK = 3200

SC_CHUNK = 3200
SC_WINDOWS = SC_CHUNK // 16
SC_NCHUNKS = N_EDGES // SC_CHUNK
FSLICE = 4


def _encode_body(x_ref, we_ref, be_ref, wm1_ref, h_ref, gt_ref):
    h = jnp.maximum(x_ref[...] @ we_ref[...].T + be_ref[...], 0.0)
    h_ref[...] = h
    gt_ref[...] = wm1_ref[...] @ h.T


def _encode(x, We, be, Wm1):
    grid = (N_PAD // NODE_BLK,)
    return pl.pallas_call(
        _encode_body,
        grid=grid,
        in_specs=[
            pl.BlockSpec((NODE_BLK, HIDDEN), lambda i: (i, 0)),
            pl.BlockSpec((HIDDEN, HIDDEN), lambda i: (0, 0)),
            pl.BlockSpec((1, HIDDEN), lambda i: (0, 0)),
            pl.BlockSpec((HIDDEN, HIDDEN), lambda i: (0, 0)),
        ],
        out_specs=[
            pl.BlockSpec((NODE_BLK, HIDDEN), lambda i: (i, 0)),
            pl.BlockSpec((HIDDEN, NODE_BLK), lambda i: (0, i)),
        ],
        out_shape=[
            jax.ShapeDtypeStruct((N_PAD, HIDDEN), jnp.float32),
            jax.ShapeDtypeStruct((HIDDEN, N_PAD), jnp.float32),
        ],
    )(x, We, be.reshape(1, HIDDEN), Wm1)


def _edge_linear_body(ea_ref, wm2_ref, et_ref):
    et_ref[...] = wm2_ref[...] @ ea_ref[...].T


def _edge_linear(edge_attr, Wm2):
    grid = (N_EDGES // EDGE_BLK,)
    return pl.pallas_call(
        _edge_linear_body,
        grid=grid,
        in_specs=[
            pl.BlockSpec((EDGE_BLK, EDGE_IN), lambda i: (i, 0)),
            pl.BlockSpec((HIDDEN, EDGE_IN), lambda i: (0, 0)),
        ],
        out_specs=pl.BlockSpec((HIDDEN, EDGE_BLK), lambda i: (0, i)),
        out_shape=jax.ShapeDtypeStruct((HIDDEN, N_EDGES), jnp.float32),
    )(edge_attr, Wm2)


def _update_body(h_ref, at_ref, wu1_ref, wu2_ref, bm_ref, bu_ref, wd_ref,
                 bd_ref, o_ref):
    a = at_ref[...].T
    a = jnp.where(a == jnp.inf, 0.0, a + bm_ref[...])
    u = h_ref[...] @ wu1_ref[...].T + a @ wu2_ref[...].T + bu_ref[...]
    d = jnp.sum(u * wd_ref[...], axis=1, keepdims=True) + bd_ref[...]
    o_ref[...] = jax.nn.sigmoid(d)


def _update_decode(h, aggT, Wu1, Wu2, bm, bu, Wd, bd):
    grid = (N_PAD // NODE_BLK,)
    return pl.pallas_call(
        _update_body,
        grid=grid,
        in_specs=[
            pl.BlockSpec((NODE_BLK, HIDDEN), lambda i: (i, 0)),
            pl.BlockSpec((HIDDEN, NODE_BLK), lambda i: (0, i)),
            pl.BlockSpec((HIDDEN, HIDDEN), lambda i: (0, 0)),
            pl.BlockSpec((HIDDEN, HIDDEN), lambda i: (0, 0)),
            pl.BlockSpec((1, HIDDEN), lambda i: (0, 0)),
            pl.BlockSpec((1, HIDDEN), lambda i: (0, 0)),
            pl.BlockSpec((1, HIDDEN), lambda i: (0, 0)),
            pl.BlockSpec((1, 1), lambda i: (0, 0)),
        ],
        out_specs=pl.BlockSpec((NODE_BLK, 1), lambda i: (i, 0)),
        out_shape=jax.ShapeDtypeStruct((N_PAD, 1), jnp.float32),
    )(h, aggT, Wu1, Wu2, bm.reshape(1, HIDDEN), bu.reshape(1, HIDDEN),
      Wd.reshape(1, HIDDEN), bd.reshape(1, 1))


def _sc_aggregate(src, dst, gT, eT):
    mesh = plsc.VectorSubcoreMesh(core_axis_name="c", subcore_axis_name="s")

    @functools.partial(
        pl.kernel,
        mesh=mesh,
        compiler_params=pltpu.CompilerParams(needs_layout_passes=False),
        out_type=jax.ShapeDtypeStruct((HIDDEN, N_PAD), jnp.float32),
        scratch_types=[
            pltpu.VMEM((FSLICE * N_PAD,), jnp.float32),
            pltpu.VMEM((FSLICE * N_PAD,), jnp.float32),
            pltpu.VMEM((2 * SC_CHUNK,), jnp.int32),
            pltpu.VMEM((2 * SC_CHUNK,), jnp.int32),
            pltpu.VMEM((2 * FSLICE * SC_CHUNK,), jnp.float32),
            pltpu.VMEM((32,), jnp.int32),
            pltpu.SemaphoreType.DMA,
            pltpu.SemaphoreType.DMA,
            pltpu.SemaphoreType.DMA,
            pltpu.SemaphoreType.DMA,
            pltpu.SemaphoreType.DMA,
            pltpu.SemaphoreType.DMA,
        ],
    )
    def agg(src_hbm, dst_hbm, gt_hbm, et_hbm, out_hbm, g_v, acc_v, s_v, d_v,
            e_v, pad_v, *sems):
        w = lax.axis_index("s") * 2 + lax.axis_index("c")
        frow = w * FSLICE

        iota = lax.iota(jnp.int32, 16)
        inf16 = jnp.full((16,), jnp.inf, jnp.float32)
        lane_sel = [iota == r for r in range(16)]

        def fire(chunk, slot):
            base = chunk * SC_CHUNK
            pltpu.async_copy(src_hbm.at[pl.ds(base, SC_CHUNK)],
                             s_v.at[pl.ds(slot * SC_CHUNK, SC_CHUNK)],
                             sems[3 * slot])
            pltpu.async_copy(dst_hbm.at[pl.ds(base, SC_CHUNK)],
                             d_v.at[pl.ds(slot * SC_CHUNK, SC_CHUNK)],
                             sems[3 * slot + 1])
            for r in range(FSLICE):
                pltpu.async_copy(
                    et_hbm.at[frow + r, pl.ds(base, SC_CHUNK)],
                    e_v.at[pl.ds((2 * r + slot) * SC_CHUNK, SC_CHUNK)],
                    sems[3 * slot + 2])

        def drain(slot):
            pltpu.make_async_copy(
                src_hbm.at[pl.ds(0, SC_CHUNK)],
                s_v.at[pl.ds(slot * SC_CHUNK, SC_CHUNK)],
                sems[3 * slot]).wait()
            pltpu.make_async_copy(
                dst_hbm.at[pl.ds(0, SC_CHUNK)],
                d_v.at[pl.ds(slot * SC_CHUNK, SC_CHUNK)],
                sems[3 * slot + 1]).wait()
            for r in range(FSLICE):
                pltpu.make_async_copy(
                    et_hbm.at[frow + r, pl.ds(0, SC_CHUNK)],
                    e_v.at[pl.ds((2 * r + slot) * SC_CHUNK, SC_CHUNK)],
                    sems[3 * slot + 2]).wait()

        for r in range(FSLICE):
            pltpu.sync_copy(gt_hbm.at[frow + r, :],
                            g_v.at[pl.ds(r * N_PAD, N_PAD)])

        def init_body(i, _):
            acc_v[pl.ds(i * 16, 16)] = inf16
            return 0

        lax.fori_loop(0, FSLICE * N_PAD // 16, init_body, 0)
        pad_v[pl.ds(16, 16)] = jnp.full((16,), -1, jnp.int32)

        fire(0, 0)
        fire(1, 1)

        def window(slot, tb):
            s16 = s_v[pl.ds(slot * SC_CHUNK + tb, 16)]
            d16 = d_v[pl.ds(slot * SC_CHUNK + tb, 16)]
            srt, _u = plsc.sort_key_val(d16, d16)
            pad_v[pl.ds(0, 16)] = srt
            sh = plsc.load_gather(pad_v, [iota + 1])
            ndup = plsc.all_reduce_population_count(srt == sh)

            ms, iads = [], []
            for f in range(FSLICE):
                ev = e_v[pl.ds((2 * f + slot) * SC_CHUNK + tb, 16)]
                gv = plsc.load_gather(g_v, [s16 + f * N_PAD])
                iads.append(d16 + f * N_PAD)
                ms.append(gv + ev)

            def fast():
                for f in range(FSLICE):
                    av = plsc.load_gather(acc_v, [iads[f]])
                    plsc.store_scatter(acc_v, [iads[f]],
                                       jnp.minimum(av, ms[f]))

            def slow():
                for f in range(FSLICE):
                    for r in range(16):
                        av = plsc.load_gather(acc_v, [iads[f]])
                        plsc.store_scatter(acc_v, [iads[f]],
                                           jnp.minimum(av, ms[f]),
                                           mask=lane_sel[r])

            lax.cond(jnp.max(ndup) > 0, slow, fast)

        def super_body(k0, _):
            for b in range(2):
                drain(b)

                def win_body(t2, _):
                    window(b, t2 * 32)
                    window(b, t2 * 32 + 16)
                    return 0

                lax.fori_loop(0, SC_WINDOWS // 2, win_body, 0)

                def refire():
                    fire(2 * k0 + b + 2, b)

                lax.cond(2 * k0 + b + 2 < SC_NCHUNKS, refire, lambda: None)
            return 0

        lax.fori_loop(0, SC_NCHUNKS // 2, super_body, 0)

        for r in range(FSLICE):
            pltpu.sync_copy(acc_v.at[pl.ds(r * N_PAD, N_PAD)],
                            out_hbm.at[frow + r, :])

    return agg(src, dst, gT, eT)


def kernel(x, edge_index, edge_attr, We, be, Wm, bm, Wu, bu, Wd, bd):
    src = edge_index[0].astype(jnp.int32)
    dst = edge_index[1].astype(jnp.int32)
    Wm1 = Wm[:, :HIDDEN]
    Wm2 = Wm[:, HIDDEN:]
    Wu1 = Wu[:, :HIDDEN]
    Wu2 = Wu[:, HIDDEN:]

    x_pad = jnp.pad(x, ((0, N_PAD - N_NODES), (0, 0)))
    h, gT = _encode(x_pad, We, be, Wm1)
    eT = _edge_linear(edge_attr, Wm2)
    aggT = _sc_aggregate(src, dst, gT, eT)
    out = _update_decode(h, aggT, Wu1, Wu2, bm, bu, Wd, bd)
    return out[:N_NODES]
